# baseline (speedup 1.0000x reference)
import functools

import jax
import jax.numpy as jnp
from jax import lax
from jax.experimental import pallas as pl
from jax.experimental.pallas import tpu as pltpu
from jax.experimental.pallas import tpu_sc as plsc

NUM_CORES = 2
NUM_SUBCORES = 16
NUM_WORKERS = NUM_CORES * NUM_SUBCORES
NBUF = 2
LANES = 16


@functools.lru_cache(maxsize=None)
def _make_kernel(S: int, R: int, V: int, D: int):
    assert S % NUM_WORKERS == 0 and D % LANES == 0
    W = S // NUM_WORKERS
    n_chunks = R

    mesh = plsc.VectorSubcoreMesh(
        core_axis_name="c", subcore_axis_name="s", num_cores=NUM_CORES,
        num_subcores=NUM_SUBCORES)

    @functools.partial(
        pl.kernel,
        out_type=jax.ShapeDtypeStruct((R, D, S), jnp.float32),
        mesh=mesh,
        scratch_types=[
            [pltpu.VMEM((W,), jnp.int32) for _ in range(NBUF)],
            [pltpu.VMEM((W, D), jnp.float32) for _ in range(NBUF)],
            [pltpu.VMEM((D, W), jnp.float32) for _ in range(NBUF)],
            [pltpu.SemaphoreType.DMA for _ in range(NBUF)],
            [pltpu.SemaphoreType.DMA for _ in range(NBUF)],
        ],
        compiler_params=pltpu.CompilerParams(use_tc_tiling_on_sc=False,
                                             needs_layout_passes=False),
    )
    def gather_kernel(tok_hbm, table_hbm, out_hbm, idx_v, rows_v, trows_v,
                      gsem, wsem):
        wid = lax.axis_index("s") * NUM_CORES + lax.axis_index("c")
        col0 = wid * W

        def load_and_fire(i, b):
            pltpu.sync_copy(tok_hbm.at[pl.ds(i * S + col0, W)], idx_v[b])
            pltpu.async_copy(table_hbm.at[idx_v[b]], rows_v[b], gsem[b])

        def wait_gather(b):
            pltpu.make_async_copy(table_hbm.at[idx_v[b]], rows_v[b],
                                  gsem[b]).wait()

        def fire_write(i, b):
            pltpu.async_copy(trows_v[b], out_hbm.at[i, :, pl.ds(col0, W)],
                             wsem[b])

        def wait_write(i, b):
            pltpu.make_async_copy(trows_v[b],
                                  out_hbm.at[i, :, pl.ds(col0, W)],
                                  wsem[b]).wait()

        def transpose(b):
            def grp_body(j0, carry):
                j_vec = j0 * LANES + lax.iota(jnp.int32, LANES)
                for d in range(D):
                    d_vec = jnp.full((LANES,), d, jnp.int32)
                    vals = plsc.load_gather(rows_v[b], [j_vec, d_vec])
                    trows_v[b][d, pl.ds(j0 * LANES, LANES)] = vals
                return carry

            lax.fori_loop(0, W // LANES, grp_body, 0)

        for b in range(NBUF):
            load_and_fire(b, b)

        for b in range(NBUF):
            wait_gather(b)
            transpose(b)
            load_and_fire(b + NBUF, b)
            fire_write(b, b)

        def group_body(g, carry):
            for b in range(NBUF):
                i = NBUF * g + b
                wait_gather(b)
                wait_write(i - NBUF, b)
                transpose(b)
                load_and_fire(i + NBUF, b)
                fire_write(i, b)
            return carry

        lax.fori_loop(1, n_chunks // NBUF - 1, group_body, 0)

        for b in range(NBUF):
            i = n_chunks - NBUF + b
            wait_gather(b)
            wait_write(i - NBUF, b)
            transpose(b)
            fire_write(i, b)
        for b in range(NBUF):
            i = n_chunks - NBUF + b
            wait_write(i, b)

    return gather_kernel


def kernel(token_ids, weight):
    S, R = token_ids.shape
    V, D = weight.shape
    flat = jnp.reshape(jnp.transpose(token_ids), (S * R,)).astype(jnp.int32)
    out = _make_kernel(S, R, V, D)(flat, weight)
    return jnp.transpose(out, (2, 0, 1))


# speedup vs baseline: 1.9464x; 1.9464x over previous
import functools

import jax
import jax.numpy as jnp
from jax import lax
from jax.experimental import pallas as pl
from jax.experimental.pallas import tpu as pltpu
from jax.experimental.pallas import tpu_sc as plsc

NUM_CORES = 2
NUM_SUBCORES = 16
NUM_WORKERS = NUM_CORES * NUM_SUBCORES
NBUF = 2
LANES = 16


@functools.lru_cache(maxsize=None)
def _make_kernel(S: int, R: int, V: int, D: int):
    assert S % NUM_WORKERS == 0 and D % LANES == 0
    W = S // NUM_WORKERS
    n_chunks = R

    mesh = plsc.VectorSubcoreMesh(
        core_axis_name="c", subcore_axis_name="s", num_cores=NUM_CORES,
        num_subcores=NUM_SUBCORES)

    @functools.partial(
        pl.kernel,
        out_type=jax.ShapeDtypeStruct((R, D, S), jnp.float32),
        mesh=mesh,
        scratch_types=[
            [pltpu.VMEM((W,), jnp.int32) for _ in range(NBUF)],
            [pltpu.VMEM((W, D), jnp.float32) for _ in range(NBUF)],
            [pltpu.VMEM((D, W + 1), jnp.float32) for _ in range(NBUF)],
            [pltpu.SemaphoreType.DMA for _ in range(NBUF)],
            [pltpu.SemaphoreType.DMA for _ in range(NBUF)],
        ],
        compiler_params=pltpu.CompilerParams(use_tc_tiling_on_sc=False,
                                             needs_layout_passes=False),
    )
    def gather_kernel(tok_hbm, table_hbm, out_hbm, idx_v, rows_v, trows_v,
                      gsem, wsem):
        wid = lax.axis_index("s") * NUM_CORES + lax.axis_index("c")
        col0 = wid * W

        def load_and_fire(i, b):
            pltpu.sync_copy(tok_hbm.at[pl.ds(i * S + col0, W)], idx_v[b])
            pltpu.async_copy(table_hbm.at[idx_v[b]], rows_v[b], gsem[b])

        def wait_gather(b):
            pltpu.make_async_copy(table_hbm.at[idx_v[b]], rows_v[b],
                                  gsem[b]).wait()

        def fire_write(i, b):
            pltpu.async_copy(trows_v[b].at[:, pl.ds(0, W)],
                             out_hbm.at[i, :, pl.ds(col0, W)], wsem[b])

        def wait_write(i, b):
            pltpu.make_async_copy(trows_v[b].at[:, pl.ds(0, W)],
                                  out_hbm.at[i, :, pl.ds(col0, W)],
                                  wsem[b]).wait()

        d_iota = lax.iota(jnp.int32, LANES)

        def transpose(b):
            def grp_body(j0, carry):
                base = j0 * 8
                for u in range(8):
                    j = base + u
                    j_vec = jnp.full((LANES,), j, jnp.int32)
                    for d0 in range(0, D, LANES):
                        vals = rows_v[b][j, pl.ds(d0, LANES)]
                        plsc.store_scatter(trows_v[b],
                                           [d_iota + d0, j_vec], vals)
                return carry

            lax.fori_loop(0, W // 8, grp_body, 0)

        for b in range(NBUF):
            load_and_fire(b, b)

        for b in range(NBUF):
            wait_gather(b)
            transpose(b)
            load_and_fire(b + NBUF, b)
            fire_write(b, b)

        def group_body(g, carry):
            for b in range(NBUF):
                i = NBUF * g + b
                wait_gather(b)
                wait_write(i - NBUF, b)
                transpose(b)
                load_and_fire(i + NBUF, b)
                fire_write(i, b)
            return carry

        lax.fori_loop(1, n_chunks // NBUF - 1, group_body, 0)

        for b in range(NBUF):
            i = n_chunks - NBUF + b
            wait_gather(b)
            wait_write(i - NBUF, b)
            transpose(b)
            fire_write(i, b)
        for b in range(NBUF):
            i = n_chunks - NBUF + b
            wait_write(i, b)

    return gather_kernel


def kernel(token_ids, weight):
    S, R = token_ids.shape
    V, D = weight.shape
    flat = jnp.reshape(jnp.transpose(token_ids), (S * R,)).astype(jnp.int32)
    out = _make_kernel(S, R, V, D)(flat, weight)
    return jnp.transpose(out, (2, 0, 1))


# speedup vs baseline: 2.6479x; 1.3604x over previous
import functools

import jax
import jax.numpy as jnp
from jax import lax
from jax.experimental import pallas as pl
from jax.experimental.pallas import tpu as pltpu
from jax.experimental.pallas import tpu_sc as plsc

NUM_CORES = 2
NUM_SUBCORES = 16
NUM_WORKERS = NUM_CORES * NUM_SUBCORES
NBUF = 2
LANES = 16


@functools.lru_cache(maxsize=None)
def _make_kernel(S: int, R: int, V: int, D: int):
    assert S % NUM_WORKERS == 0 and D % LANES == 0
    W = S // NUM_WORKERS
    n_chunks = R

    mesh = plsc.VectorSubcoreMesh(
        core_axis_name="c", subcore_axis_name="s", num_cores=NUM_CORES,
        num_subcores=NUM_SUBCORES)

    @functools.partial(
        pl.kernel,
        out_type=jax.ShapeDtypeStruct((R, D, S), jnp.float32),
        mesh=mesh,
        scratch_types=[
            [pltpu.VMEM((W,), jnp.int32) for _ in range(NBUF)],
            [pltpu.VMEM((W, D), jnp.float32) for _ in range(NBUF)],
            [pltpu.VMEM((D, W + 1), jnp.float32) for _ in range(NBUF)],
            [pltpu.SemaphoreType.DMA for _ in range(NBUF)],
            [pltpu.SemaphoreType.DMA for _ in range(NBUF)],
        ],
        compiler_params=pltpu.CompilerParams(use_tc_tiling_on_sc=False,
                                             needs_layout_passes=False),
    )
    def gather_kernel(tok_hbm, table_hbm, out_hbm, idx_v, rows_v, trows_v,
                      gsem, wsem):
        wid = lax.axis_index("s") * NUM_CORES + lax.axis_index("c")
        col0 = wid * W

        def load_and_fire(i, b):
            pltpu.sync_copy(tok_hbm.at[pl.ds(i * S + col0, W)], idx_v[b])
            pltpu.async_copy(table_hbm.at[idx_v[b]], rows_v[b], gsem[b])

        def wait_gather(b):
            pltpu.make_async_copy(table_hbm.at[idx_v[b]], rows_v[b],
                                  gsem[b]).wait()

        def fire_write(i, b):
            pltpu.async_copy(trows_v[b].at[:, pl.ds(0, W)],
                             out_hbm.at[i, :, pl.ds(col0, W)], wsem[b])

        def wait_write(i, b):
            pltpu.make_async_copy(trows_v[b].at[:, pl.ds(0, W)],
                                  out_hbm.at[i, :, pl.ds(col0, W)],
                                  wsem[b]).wait()

        d_iota = lax.iota(jnp.int32, LANES)

        def transpose(b):
            @plsc.parallel_loop(0, W, step=1, unroll=8)
            def _(j):
                j_vec = jnp.full((LANES,), j, jnp.int32)
                for d0 in range(0, D, LANES):
                    vals = rows_v[b][j, pl.ds(d0, LANES)]
                    plsc.store_scatter(trows_v[b], [d_iota + d0, j_vec],
                                       vals)

        for b in range(NBUF):
            load_and_fire(b, b)

        for b in range(NBUF):
            wait_gather(b)
            transpose(b)
            load_and_fire(b + NBUF, b)
            fire_write(b, b)

        def group_body(g, carry):
            for b in range(NBUF):
                i = NBUF * g + b
                wait_gather(b)
                wait_write(i - NBUF, b)
                transpose(b)
                load_and_fire(i + NBUF, b)
                fire_write(i, b)
            return carry

        lax.fori_loop(1, n_chunks // NBUF - 1, group_body, 0)

        for b in range(NBUF):
            i = n_chunks - NBUF + b
            wait_gather(b)
            wait_write(i - NBUF, b)
            transpose(b)
            fire_write(i, b)
        for b in range(NBUF):
            i = n_chunks - NBUF + b
            wait_write(i, b)

    return gather_kernel


def kernel(token_ids, weight):
    S, R = token_ids.shape
    V, D = weight.shape
    flat = jnp.reshape(jnp.transpose(token_ids), (S * R,)).astype(jnp.int32)
    out = _make_kernel(S, R, V, D)(flat, weight)
    return jnp.transpose(out, (2, 0, 1))


# speedup vs baseline: 3.7621x; 1.4208x over previous
import functools

import jax
import jax.numpy as jnp
from jax import lax
from jax.experimental import pallas as pl
from jax.experimental.pallas import tpu as pltpu
from jax.experimental.pallas import tpu_sc as plsc

NUM_CORES = 2
NUM_SUBCORES = 16
NUM_WORKERS = NUM_CORES * NUM_SUBCORES
NBUF = 2
LANES = 16
TILE_R = 8
TILE_C = 128


@functools.lru_cache(maxsize=None)
def _make_kernel(S: int, R: int, V: int, D: int):
    assert S % (NUM_WORKERS * TILE_C) == 0 and D % LANES == 0
    assert D % TILE_R == 0
    W = S // NUM_WORKERS
    NCT = W // TILE_C
    NDT = D // TILE_R
    PADC = TILE_C + 1
    n_chunks = R

    mesh = plsc.VectorSubcoreMesh(
        core_axis_name="c", subcore_axis_name="s", num_cores=NUM_CORES,
        num_subcores=NUM_SUBCORES)

    @functools.partial(
        pl.kernel,
        out_type=jax.ShapeDtypeStruct((R, NDT, S // TILE_C, TILE_R, TILE_C),
                                      jnp.float32),
        mesh=mesh,
        scratch_types=[
            [pltpu.VMEM((W,), jnp.int32) for _ in range(NBUF)],
            [pltpu.VMEM((W, D), jnp.float32) for _ in range(NBUF)],
            [pltpu.VMEM((NDT, NCT, TILE_R, PADC), jnp.float32)
             for _ in range(NBUF)],
            [pltpu.SemaphoreType.DMA for _ in range(NBUF)],
            [pltpu.SemaphoreType.DMA for _ in range(NBUF)],
        ],
        compiler_params=pltpu.CompilerParams(use_tc_tiling_on_sc=False,
                                             needs_layout_passes=False),
    )
    def gather_kernel(tok_hbm, table_hbm, out_hbm, idx_v, rows_v, trows_v,
                      gsem, wsem):
        wid = lax.axis_index("s") * NUM_CORES + lax.axis_index("c")
        col0 = wid * W
        ct0 = wid * NCT

        def load_and_fire(i, b):
            pltpu.sync_copy(tok_hbm.at[pl.ds(i * S + col0, W)], idx_v[b])
            pltpu.async_copy(table_hbm.at[idx_v[b]], rows_v[b], gsem[b])

        def wait_gather(b):
            pltpu.make_async_copy(table_hbm.at[idx_v[b]], rows_v[b],
                                  gsem[b]).wait()

        def fire_write(i, b):
            pltpu.async_copy(trows_v[b].at[:, :, :, pl.ds(0, TILE_C)],
                             out_hbm.at[i, :, pl.ds(ct0, NCT), :, :],
                             wsem[b])

        def wait_write(i, b):
            pltpu.make_async_copy(trows_v[b].at[:, :, :, pl.ds(0, TILE_C)],
                                  out_hbm.at[i, :, pl.ds(ct0, NCT), :, :],
                                  wsem[b]).wait()

        d_iota = lax.iota(jnp.int32, LANES)

        def transpose(b):
            @plsc.parallel_loop(0, W, step=1, unroll=8)
            def _(j):
                ct_vec = jnp.full((LANES,), j // TILE_C, jnp.int32)
                ic_vec = jnp.full((LANES,), j % TILE_C, jnp.int32)
                for d0 in range(0, D, LANES):
                    d_vec = d_iota + d0
                    vals = rows_v[b][j, pl.ds(d0, LANES)]
                    plsc.store_scatter(
                        trows_v[b],
                        [d_vec // TILE_R, ct_vec, d_vec % TILE_R, ic_vec],
                        vals)

        for b in range(NBUF):
            load_and_fire(b, b)

        for b in range(NBUF):
            wait_gather(b)
            transpose(b)
            load_and_fire(b + NBUF, b)
            fire_write(b, b)

        def group_body(g, carry):
            for b in range(NBUF):
                i = NBUF * g + b
                wait_gather(b)
                wait_write(i - NBUF, b)
                transpose(b)
                load_and_fire(i + NBUF, b)
                fire_write(i, b)
            return carry

        lax.fori_loop(1, n_chunks // NBUF - 1, group_body, 0)

        for b in range(NBUF):
            i = n_chunks - NBUF + b
            wait_gather(b)
            wait_write(i - NBUF, b)
            transpose(b)
            fire_write(i, b)
        for b in range(NBUF):
            i = n_chunks - NBUF + b
            wait_write(i, b)

    return gather_kernel


def kernel(token_ids, weight):
    S, R = token_ids.shape
    V, D = weight.shape
    flat = jnp.reshape(jnp.transpose(token_ids), (S * R,)).astype(jnp.int32)
    out5 = _make_kernel(S, R, V, D)(flat, weight)
    return jnp.reshape(jnp.transpose(out5, (2, 4, 0, 1, 3)), (S, R, D))


# speedup vs baseline: 3.8667x; 1.0278x over previous
import functools

import jax
import jax.numpy as jnp
from jax import lax
from jax.experimental import pallas as pl
from jax.experimental.pallas import tpu as pltpu
from jax.experimental.pallas import tpu_sc as plsc

NUM_CORES = 2
NUM_SUBCORES = 16
NUM_WORKERS = NUM_CORES * NUM_SUBCORES
NBUF = 2
LANES = 16
TILE_R = 8
TILE_C = 128


@functools.lru_cache(maxsize=None)
def _make_kernel(S: int, R: int, V: int, D: int):
    assert S % (NUM_WORKERS * TILE_C) == 0 and D % LANES == 0
    assert D % TILE_R == 0
    W = S // NUM_WORKERS
    NCT = W // TILE_C
    NDT = D // TILE_R
    PADC = TILE_C + 1
    n_chunks = R

    mesh = plsc.VectorSubcoreMesh(
        core_axis_name="c", subcore_axis_name="s", num_cores=NUM_CORES,
        num_subcores=NUM_SUBCORES)

    @functools.partial(
        pl.kernel,
        out_type=jax.ShapeDtypeStruct((R, NDT, S // TILE_C, TILE_R, TILE_C),
                                      jnp.float32),
        mesh=mesh,
        scratch_types=[
            [pltpu.VMEM((W,), jnp.int32) for _ in range(NBUF)],
            [pltpu.VMEM((W, D), jnp.float32) for _ in range(NBUF)],
            [pltpu.VMEM((D // LANES, NCT, LANES, PADC), jnp.float32)
             for _ in range(NBUF)],
            [pltpu.SemaphoreType.DMA for _ in range(NBUF)],
            [pltpu.SemaphoreType.DMA for _ in range(NBUF)],
        ],
        compiler_params=pltpu.CompilerParams(use_tc_tiling_on_sc=False,
                                             needs_layout_passes=False),
    )
    def gather_kernel(tok_hbm, table_hbm, out_hbm, idx_v, rows_v, trows_v,
                      gsem, wsem):
        wid = lax.axis_index("s") * NUM_CORES + lax.axis_index("c")
        col0 = wid * W
        ct0 = wid * NCT

        def load_and_fire(i, b):
            pltpu.sync_copy(tok_hbm.at[pl.ds(i * S + col0, W)], idx_v[b])
            pltpu.async_copy(table_hbm.at[idx_v[b]], rows_v[b], gsem[b])

        def wait_gather(b):
            pltpu.make_async_copy(table_hbm.at[idx_v[b]], rows_v[b],
                                  gsem[b]).wait()

        def _write_pairs(i, b):
            for dt in range(NDT):
                src = trows_v[b].at[dt // 2, :,
                                    pl.ds((dt % 2) * TILE_R, TILE_R),
                                    pl.ds(0, TILE_C)]
                dst = out_hbm.at[i, dt, pl.ds(ct0, NCT), :, :]
                yield src, dst

        def fire_write(i, b):
            for src, dst in _write_pairs(i, b):
                pltpu.async_copy(src, dst, wsem[b])

        def wait_write(i, b):
            for src, dst in _write_pairs(i, b):
                pltpu.make_async_copy(src, dst, wsem[b]).wait()

        d_iota = lax.iota(jnp.int32, LANES)

        def transpose(b):
            @plsc.parallel_loop(0, W, step=1, unroll=8)
            def _(j):
                ct_vec = jnp.full((LANES,), j // TILE_C, jnp.int32)
                ic_vec = jnp.full((LANES,), j % TILE_C, jnp.int32)
                for g in range(D // LANES):
                    g_vec = jnp.full((LANES,), g, jnp.int32)
                    vals = rows_v[b][j, pl.ds(g * LANES, LANES)]
                    plsc.store_scatter(
                        trows_v[b], [g_vec, ct_vec, d_iota, ic_vec], vals)

        for b in range(NBUF):
            load_and_fire(b, b)

        for b in range(NBUF):
            wait_gather(b)
            transpose(b)
            load_and_fire(b + NBUF, b)
            fire_write(b, b)

        def group_body(g, carry):
            for b in range(NBUF):
                i = NBUF * g + b
                wait_gather(b)
                wait_write(i - NBUF, b)
                transpose(b)
                load_and_fire(i + NBUF, b)
                fire_write(i, b)
            return carry

        lax.fori_loop(1, n_chunks // NBUF - 1, group_body, 0)

        for b in range(NBUF):
            i = n_chunks - NBUF + b
            wait_gather(b)
            wait_write(i - NBUF, b)
            transpose(b)
            fire_write(i, b)
        for b in range(NBUF):
            i = n_chunks - NBUF + b
            wait_write(i, b)

    return gather_kernel


def kernel(token_ids, weight):
    S, R = token_ids.shape
    V, D = weight.shape
    flat = jnp.reshape(jnp.transpose(token_ids), (S * R,)).astype(jnp.int32)
    out5 = _make_kernel(S, R, V, D)(flat, weight)
    return jnp.reshape(jnp.transpose(out5, (2, 4, 0, 1, 3)), (S, R, D))
